# Initial kernel scaffold; baseline (speedup 1.0000x reference)
import jax
import jax.numpy as jnp
from jax.experimental import pallas as pl


def kernel(x, edge_index, W1, b1, W2, b2, W3, b3):
    raise NotImplementedError("write your pallas kernel here")


# speedup vs baseline: 5.4179x; 5.4179x over previous
import functools

import jax
import jax.numpy as jnp
from jax import lax
from jax.experimental import pallas as pl
from jax.experimental.pallas import tpu as pltpu
import jax.experimental.pallas.tpu_sc as plsc

NC = 2
NS = 16
NW = NC * NS
CH = 128
DW = 16

_MESH = dict(core_axis_name="c", subcore_axis_name="s", num_cores=NC,
             num_subcores=NS)


def _fill2d(ref, value, rows, cols):
    vec = jnp.full((16,), value, jnp.float32)

    @pl.loop(0, rows)
    def _(r):
        @pl.loop(0, cols // 16)
        def _(j):
            ref[r, pl.ds(j * 16, 16)] = vec


def _deg_body(ew, nf, rem, rpt, src_hbm, dst_hbm, dout_hbm, din_hbm,
              dout_sh, din_sh, ones_v, zero_v, sidx_v, didx_v,
              sidx_r, didx_r):
    c = lax.axis_index("c")
    s = lax.axis_index("s")
    wid = c * NS + s
    _fill2d(ones_v, 1.0, CH, DW)
    _fill2d(zero_v, 0.0, rpt, DW)
    pltpu.sync_copy(zero_v, dout_sh.at[pl.ds(s * rpt, rpt)])
    pltpu.sync_copy(zero_v, din_sh.at[pl.ds(s * rpt, rpt)])
    plsc.subcore_barrier()
    base = wid * ew

    @pl.loop(0, nf)
    def _(i):
        off = base + i * CH
        pltpu.sync_copy(src_hbm.at[pl.ds(off, CH)], sidx_v)
        pltpu.sync_copy(dst_hbm.at[pl.ds(off, CH)], didx_v)
        pltpu.sync_copy(ones_v, dout_sh.at[sidx_v], add=True)
        pltpu.sync_copy(ones_v, din_sh.at[didx_v], add=True)

    if rem:
        offr = base + nf * CH
        pltpu.sync_copy(src_hbm.at[pl.ds(offr, rem)], sidx_r)
        pltpu.sync_copy(dst_hbm.at[pl.ds(offr, rem)], didx_r)
        pltpu.sync_copy(ones_v.at[pl.ds(0, rem)], dout_sh.at[sidx_r], add=True)
        pltpu.sync_copy(ones_v.at[pl.ds(0, rem)], din_sh.at[didx_r], add=True)
    plsc.subcore_barrier()
    pltpu.sync_copy(dout_sh.at[pl.ds(s * rpt, rpt)],
                    dout_hbm.at[c, pl.ds(s * rpt, rpt)])
    pltpu.sync_copy(din_sh.at[pl.ds(s * rpt, rpt)],
                    din_hbm.at[c, pl.ds(s * rpt, rpt)])


def _scatter_body(ew, nf, rem, rpt, h_hbm, src_hbm, dst_hbm, out_hbm,
                  agg_sh, rows_v, zero_v, sidx_v, didx_v, sidx_r, didx_r,
                  rows_r):
    c = lax.axis_index("c")
    s = lax.axis_index("s")
    wid = c * NS + s
    _fill2d(zero_v, 0.0, 64, 128)

    @pl.loop(0, rpt // 64)
    def _(k):
        pltpu.sync_copy(zero_v, agg_sh.at[pl.ds(s * rpt + k * 64, 64)])

    plsc.subcore_barrier()
    base = wid * ew

    @pl.loop(0, nf)
    def _(i):
        off = base + i * CH
        pltpu.sync_copy(src_hbm.at[pl.ds(off, CH)], sidx_v)
        pltpu.sync_copy(dst_hbm.at[pl.ds(off, CH)], didx_v)
        pltpu.sync_copy(h_hbm.at[sidx_v], rows_v)
        pltpu.sync_copy(rows_v, agg_sh.at[didx_v], add=True)

    if rem:
        offr = base + nf * CH
        pltpu.sync_copy(src_hbm.at[pl.ds(offr, rem)], sidx_r)
        pltpu.sync_copy(dst_hbm.at[pl.ds(offr, rem)], didx_r)
        pltpu.sync_copy(h_hbm.at[sidx_r], rows_r)
        pltpu.sync_copy(rows_r, agg_sh.at[didx_r], add=True)
    plsc.subcore_barrier()
    pltpu.sync_copy(agg_sh.at[pl.ds(s * rpt, rpt)],
                    out_hbm.at[c, pl.ds(s * rpt, rpt)])


def _make_deg(e, np_, ew, nf, rem):
    rpt = np_ // NS
    body = functools.partial(_deg_body, ew, nf, rem, rpt)
    return pl.kernel(
        body,
        out_type=(jax.ShapeDtypeStruct((NC, np_, DW), jnp.float32),
                  jax.ShapeDtypeStruct((NC, np_, DW), jnp.float32)),
        mesh=plsc.VectorSubcoreMesh(**_MESH),
        scratch_types=[
            pltpu.VMEM_SHARED((np_, DW), jnp.float32),
            pltpu.VMEM_SHARED((np_, DW), jnp.float32),
            pltpu.VMEM((CH, DW), jnp.float32),
            pltpu.VMEM((rpt, DW), jnp.float32),
            pltpu.VMEM((CH,), jnp.int32),
            pltpu.VMEM((CH,), jnp.int32),
            pltpu.VMEM((max(rem, 8),), jnp.int32),
            pltpu.VMEM((max(rem, 8),), jnp.int32),
        ],
    )


def _make_scatter(e, np_, d, ew, nf, rem):
    rpt = np_ // NS
    body = functools.partial(_scatter_body, ew, nf, rem, rpt)
    return pl.kernel(
        body,
        out_type=jax.ShapeDtypeStruct((NC, np_, d), jnp.float32),
        mesh=plsc.VectorSubcoreMesh(**_MESH),
        scratch_types=[
            pltpu.VMEM_SHARED((np_, d), jnp.float32),
            pltpu.VMEM((CH, d), jnp.float32),
            pltpu.VMEM((64, 128), jnp.float32),
            pltpu.VMEM((CH,), jnp.int32),
            pltpu.VMEM((CH,), jnp.int32),
            pltpu.VMEM((max(rem, 8),), jnp.int32),
            pltpu.VMEM((max(rem, 8),), jnp.int32),
            pltpu.VMEM((max(rem, 8), d), jnp.float32),
        ],
    )


BR = 1024


def _prep_body(dout_ref, din_ref, x_ref, soutb_ref, sinb_ref, h0_ref):
    do = dout_ref[0, :, 0:1] + dout_ref[1, :, 0:1]
    di = din_ref[0, :, 0:1] + din_ref[1, :, 0:1]
    so = lax.rsqrt(jnp.maximum(do, 1.0))
    si = lax.rsqrt(jnp.maximum(di, 1.0))
    soutb_ref[...] = jnp.broadcast_to(so, soutb_ref.shape)
    sinb_ref[...] = jnp.broadcast_to(si, sinb_ref.shape)
    h0_ref[...] = x_ref[...] * so


def _dense_act_body(p_ref, sinb_ref, soutb_ref, w_ref, b_ref, o_ref):
    agg = (p_ref[0] + p_ref[1]) * sinb_ref[...]
    h = jnp.dot(agg, w_ref[...], preferred_element_type=jnp.float32,
                precision=lax.Precision.HIGHEST) + b_ref[...]
    o_ref[...] = jnp.maximum(h, 0.0) * soutb_ref[...]


def _dense_last_body(p_ref, sinb_ref, w_ref, b_ref, o_ref):
    agg = (p_ref[0] + p_ref[1]) * sinb_ref[...]
    o_ref[...] = jnp.dot(agg, w_ref[...], preferred_element_type=jnp.float32,
                         precision=lax.Precision.HIGHEST) + b_ref[...]


def kernel(x, edge_index, W1, b1, W2, b2, W3, b3):
    n, d = x.shape
    e = edge_index.shape[1]
    np_ = ((n + 2047) // 2048) * 2048
    ew = e // NW
    nf = ew // CH
    rem = ew - nf * CH

    src = edge_index[0]
    dst = edge_index[1]
    x_p = jnp.pad(x, ((0, np_ - n), (0, 0)))

    deg_fn = _make_deg(e, np_, ew, nf, rem)
    scat_fn = _make_scatter(e, np_, d, ew, nf, rem)

    dout_p, din_p = deg_fn(src, dst)

    grid = (np_ // BR,)
    soutb, sinb, h0 = pl.pallas_call(
        _prep_body,
        grid=grid,
        in_specs=[
            pl.BlockSpec((NC, BR, DW), lambda i: (0, i, 0)),
            pl.BlockSpec((NC, BR, DW), lambda i: (0, i, 0)),
            pl.BlockSpec((BR, d), lambda i: (i, 0)),
        ],
        out_specs=[
            pl.BlockSpec((BR, d), lambda i: (i, 0)),
            pl.BlockSpec((BR, d), lambda i: (i, 0)),
            pl.BlockSpec((BR, d), lambda i: (i, 0)),
        ],
        out_shape=[jax.ShapeDtypeStruct((np_, d), jnp.float32)] * 3,
    )(dout_p, din_p, x_p)

    dense_act = pl.pallas_call(
        _dense_act_body,
        grid=grid,
        in_specs=[
            pl.BlockSpec((NC, BR, d), lambda i: (0, i, 0)),
            pl.BlockSpec((BR, d), lambda i: (i, 0)),
            pl.BlockSpec((BR, d), lambda i: (i, 0)),
            pl.BlockSpec((d, d), lambda i: (0, 0)),
            pl.BlockSpec((1, d), lambda i: (0, 0)),
        ],
        out_specs=pl.BlockSpec((BR, d), lambda i: (i, 0)),
        out_shape=jax.ShapeDtypeStruct((np_, d), jnp.float32),
    )
    dense_last = pl.pallas_call(
        _dense_last_body,
        grid=grid,
        in_specs=[
            pl.BlockSpec((NC, BR, d), lambda i: (0, i, 0)),
            pl.BlockSpec((BR, d), lambda i: (i, 0)),
            pl.BlockSpec((d, d), lambda i: (0, 0)),
            pl.BlockSpec((1, d), lambda i: (0, 0)),
        ],
        out_specs=pl.BlockSpec((BR, d), lambda i: (i, 0)),
        out_shape=jax.ShapeDtypeStruct((np_, d), jnp.float32),
    )

    b1r = b1.reshape(1, d)
    b2r = b2.reshape(1, d)
    b3r = b3.reshape(1, d)

    p = scat_fn(h0, src, dst)
    h1 = dense_act(p, sinb, soutb, W1, b1r)
    p = scat_fn(h1, src, dst)
    h2 = dense_act(p, sinb, soutb, W2, b2r)
    p = scat_fn(h2, src, dst)
    h3 = dense_last(p, sinb, W3, b3r)
    return h3[:n]
